# baseline (speedup 1.0000x reference)
import functools

import jax
import jax.numpy as jnp
from jax import lax
from jax.experimental import pallas as pl
from jax.experimental.pallas import tpu as pltpu
from jax.experimental.pallas import tpu_sc as plsc

NC = 2
NS = 16
LANES = 16


def _tc_body(pred_ref, target_ref, s_ref, e_ref, it_ref, out_ref, acc_ref,
             *, HB, N, D, G, T):
    b = pl.program_id(0)
    p = pred_ref[0]
    t = target_ref[0]
    d = p - t
    err = d * d
    ones_d = jnp.ones((D, 1), jnp.float32)
    tok = jax.lax.dot_general(
        err, ones_d, (((1,), (0,)), ((), ())),
        preferred_element_type=jnp.float32,
        precision=jax.lax.Precision.HIGHEST) * (1.0 / D)

    s = s_ref[0]
    e = e_ref[0]
    n_iota = jax.lax.broadcasted_iota(jnp.int32, (G, N), 1)
    mask = ((n_iota >= s) & (n_iota < e)).astype(jnp.float32)
    gsum = jax.lax.dot_general(
        mask, tok, (((1,), (0,)), ((), ())),
        preferred_element_type=jnp.float32,
        precision=jax.lax.Precision.HIGHEST)
    cnt = (e - s).astype(jnp.float32)
    g_err = gsum / jnp.maximum(cnt, 1.0)

    it = it_ref[0]
    maxv = jnp.max(it, axis=1, keepdims=True)
    colidx = jax.lax.broadcasted_iota(jnp.int32, (G, T), 1)
    am = jnp.min(jnp.where(it == maxv, colidx, T), axis=1, keepdims=True)
    onehot = (colidx == am).astype(jnp.float32)

    tsum = jnp.sum(onehot * g_err, axis=0, keepdims=True)
    tcnt = jnp.sum(onehot, axis=0, keepdims=True)
    part = jnp.concatenate([tsum, tcnt], axis=0)

    @pl.when(b == 0)
    def _():
        acc_ref[0:2, 0:T] = part

    @pl.when(b > 0)
    def _():
        acc_ref[0:2, 0:T] = acc_ref[0:2, 0:T] + part

    @pl.when(b == HB - 1)
    def _():
        vals = acc_ref[0:2, 0:T]
        out_ref[0:1, 0:T] = vals[0:1, :]
        out_ref[0:1, T:2 * T] = vals[1:2, :]


def _sc_main(pred_hbm, target_hbm, idx_hbm, it_hbm, out_hbm,
             pbuf0, pbuf1, tbuf0, tbuf1, idx_v, bcnt_v, gmap_v, gsum_v,
             it_v, cacc_v, row4_v, acc4_v, out_v, bst_v,
             shared_gsum, shared_acc,
             sp0, sp1, st0, st1,
             *, B, N, D, G, T, HB, NTPB, TOK, CH, NCH):
    c = lax.axis_index("c")
    s = lax.axis_index("s")
    wid = c * NS + s
    b = (B - HB) + wid // NTPB
    pos_base = (wid % NTPB) * TOK
    lanes = lax.iota(jnp.int32, LANES)
    zero16i = jnp.zeros((LANES,), jnp.int32)
    zero16f = jnp.zeros((LANES,), jnp.float32)

    CHW = CH * D

    def _issue(ch, pb, tb, sp, st):
        off = (b * N + pos_base + ch * CH) * D
        pltpu.async_copy(pred_hbm.at[pl.ds(off, CHW)], pb, sp)
        pltpu.async_copy(target_hbm.at[pl.ds(off, CHW)], tb, st)

    _issue(0, pbuf0, tbuf0, sp0, st0)
    _issue(1, pbuf1, tbuf1, sp1, st1)

    pltpu.sync_copy(idx_hbm.at[b], idx_v)

    def _zb(i, carry):
        bcnt_v[pl.ds(i * LANES, LANES)] = zero16i
        return carry
    lax.fori_loop(0, TOK // LANES, _zb, 0)

    ones16i = jnp.ones((LANES,), jnp.int32)
    for v in range((G + LANES) // LANES):
        j = v * LANES + lanes
        bv = idx_v[pl.ds(v * LANES, LANES)]
        rel = bv - pos_base
        valid = (j <= G) & (rel < TOK)
        tgt = jnp.maximum(rel, 0)
        plsc.addupdate_scatter(bcnt_v, [tgt], ones16i, mask=valid)

    def _cs(i, carry):
        x = bcnt_v[pl.ds(i * LANES, LANES)]
        cs = plsc.cumsum(x) + carry
        g = cs - 1
        g = jnp.where((g >= 0) & (g < G), g, 127)
        gmap_v[pl.ds(i * LANES, LANES)] = g
        return carry + jnp.sum(x)
    lax.fori_loop(0, TOK // LANES, _cs, jnp.int32(0))

    for k in range(128 // LANES):
        gsum_v[pl.ds(k * LANES, LANES)] = zero16f

    def _wait(pb, tb, sp, st):
        pltpu.make_async_copy(pred_hbm.at[pl.ds(0, CHW)], pb, sp).wait()
        pltpu.make_async_copy(target_hbm.at[pl.ds(0, CHW)], tb, st).wait()

    shuf_dn = lax.GatherDimensionNumbers(
        offset_dims=(), collapsed_slice_dims=(0,), start_index_map=(0,))

    def _shuf(v, idx):
        return lax.gather(v, idx[:, None], shuf_dn, slice_sizes=(1,),
                          mode=lax.GatherScatterMode.PROMISE_IN_BOUNDS)

    xor_idx = {o: lanes ^ o for o in (8, 4, 2, 1)}
    xor_msk = {o: (lanes & o) == 0 for o in (8, 4, 2, 1)}
    pi_vec = (((lanes & 1) << 3) | ((lanes & 2) << 1) |
              ((lanes & 4) >> 1) | ((lanes & 8) >> 3))

    def _combine(x, y, o):
        return jnp.where(xor_msk[o],
                         x + _shuf(x, xor_idx[o]),
                         y + _shuf(y, xor_idx[o]))

    def _fold(pb, tb, off):
        sqs = []
        for v in range(D // LANES):
            pv = pb[pl.ds(off + v * LANES, LANES)]
            tv = tb[pl.ds(off + v * LANES, LANES)]
            dv = pv - tv
            sqs.append(dv * dv)
        while len(sqs) > 1:
            sqs = [sqs[2 * k] + sqs[2 * k + 1] for k in range(len(sqs) // 2)]
        return sqs[0]

    def _compute(pb, tb, ch, bst_v):
        def _tg(tg, carry):
            for j in range(LANES // 2):
                off = (tg * LANES + 2 * j) * D
                a0 = _fold(pb, tb, off)
                a1 = _fold(pb, tb, off + D)
                bst_v[pl.ds(j * LANES, LANES)] = _combine(a0, a1, 8)
            bs_ = [bst_v[pl.ds(j * LANES, LANES)] for j in range(LANES // 2)]
            cs_ = [_combine(bs_[2 * k], bs_[2 * k + 1], 4) for k in range(4)]
            ds_ = [_combine(cs_[2 * k], cs_[2 * k + 1], 2) for k in range(2)]
            te = _combine(ds_[0], ds_[1], 1)
            g = gmap_v[pl.ds(ch * CH + tg * LANES, LANES)]
            gp = _shuf(g, pi_vec)
            plsc.addupdate_scatter(gsum_v, [gp], te)
            return carry
        lax.fori_loop(0, CH // LANES, _tg, 0)

    def _pair(i, carry):
        ch0 = 2 * i
        _wait(pbuf0, tbuf0, sp0, st0)
        _compute(pbuf0, tbuf0, ch0, bst_v)

        @pl.when(ch0 + 2 < NCH)
        def _():
            _issue(ch0 + 2, pbuf0, tbuf0, sp0, st0)

        _wait(pbuf1, tbuf1, sp1, st1)
        _compute(pbuf1, tbuf1, ch0 + 1, bst_v)

        @pl.when(ch0 + 3 < NCH)
        def _():
            _issue(ch0 + 3, pbuf1, tbuf1, sp1, st1)
        return carry
    lax.fori_loop(0, NCH // 2, _pair, 0)

    pltpu.sync_copy(gsum_v, shared_gsum.at[pl.ds(s * 128, 128)])
    plsc.subcore_barrier()

    HBC = HB // NC

    @pl.when(s < HBC)
    def _():
        bg = (B - HB) + c * HBC + s
        pltpu.sync_copy(shared_gsum.at[pl.ds((NTPB * s) * 128, NTPB * 128)],
                        row4_v)
        pltpu.sync_copy(idx_hbm.at[bg], idx_v)
        pltpu.sync_copy(it_hbm.at[bg], it_v)
        cacc_v[pl.ds(0, LANES)] = zero16f
        cacc_v[pl.ds(LANES, LANES)] = zero16f
        ones16f = jnp.ones((LANES,), jnp.float32)
        for j in range(G // LANES):
            gl = j * LANES + lanes
            st_ = plsc.load_gather(idx_v, [gl])
            en_ = plsc.load_gather(idx_v, [gl + 1])
            cnt = (en_ - st_).astype(jnp.float32)
            gs = None
            for q in range(NTPB):
                piece = row4_v[pl.ds(q * 128 + j * LANES, LANES)]
                gs = piece if gs is None else gs + piece
            ge = gs * (1.0 / D) / jnp.maximum(cnt, 1.0)
            base2 = gl * T
            m0 = plsc.load_gather(it_v, [base2])
            am = zero16i
            for tt in range(1, T):
                col = plsc.load_gather(it_v, [base2 + tt])
                better = col > m0
                am = jnp.where(better, tt, am)
                m0 = jnp.where(better, col, m0)
            plsc.addupdate_scatter(cacc_v, [am], ge)
            plsc.addupdate_scatter(cacc_v, [am + T], ones16f)
        pltpu.sync_copy(cacc_v.at[pl.ds(0, 2 * T)],
                        shared_acc.at[pl.ds(s * (2 * T), 2 * T)])

    plsc.subcore_barrier()

    @pl.when(s == 0)
    def _():
        pltpu.sync_copy(shared_acc, acc4_v)
        ts = zero16f
        tc = zero16f
        for i in range(HBC):
            ts = ts + acc4_v[pl.ds(i * (2 * T), T)]
            tc = tc + acc4_v[pl.ds(i * (2 * T) + T, T)]
        out_v[pl.ds(0, T)] = ts
        out_v[pl.ds(T, T)] = tc
        pltpu.sync_copy(out_v, out_hbm.at[c])


def _comb_body(tc_ref, sc_ref, out_ref, *, T):
    tp = tc_ref[...]
    sp = sc_ref[...]
    ts = tp[0:1, 0:T] + sp[0:1, 0:T] + sp[1:2, 0:T]
    tc = tp[0:1, T:2 * T] + sp[0:1, T:2 * T] + sp[1:2, T:2 * T]
    out_ref[...] = jnp.where(tc > 0, ts / jnp.maximum(tc, 1.0), 0.0)


def kernel(pred, target, indices, indices_type, type_names):
    B, N, D = pred.shape
    G = indices.shape[1] - 1
    T = indices_type.shape[2]
    HB = B // 2

    starts = indices[:, :-1, None]
    ends = indices[:, 1:, None]
    tc_part = pl.pallas_call(
        functools.partial(_tc_body, HB=HB, N=N, D=D, G=G, T=T),
        grid=(HB,),
        in_specs=[
            pl.BlockSpec((1, N, D), lambda b: (b, 0, 0)),
            pl.BlockSpec((1, N, D), lambda b: (b, 0, 0)),
            pl.BlockSpec((1, G, 1), lambda b: (b, 0, 0)),
            pl.BlockSpec((1, G, 1), lambda b: (b, 0, 0)),
            pl.BlockSpec((1, G, T), lambda b: (b, 0, 0)),
        ],
        out_specs=pl.BlockSpec((1, 2 * T), lambda b: (0, 0)),
        out_shape=jax.ShapeDtypeStruct((1, 2 * T), jnp.float32),
        scratch_shapes=[pltpu.VMEM((8, 128), jnp.float32)],
    )(pred, target, starts, ends, indices_type)

    NT = NC * NS
    NTPB = NT // HB
    TOK = N // NTPB
    CH = 128
    NCH = TOK // CH

    pred1 = pred.reshape(B * N * D)
    target1 = target.reshape(B * N * D)
    idx_pad = jnp.pad(indices, ((0, 0), (0, 128 - (G + 1))))
    it2 = indices_type.reshape(B, G * T)

    mesh = plsc.VectorSubcoreMesh(core_axis_name="c", subcore_axis_name="s")
    params = pltpu.CompilerParams(needs_layout_passes=False)
    sc_part = functools.partial(
        pl.kernel,
        out_type=jax.ShapeDtypeStruct((NC, 2 * T), jnp.float32),
        mesh=mesh,
        compiler_params=params,
        scratch_types=[
            pltpu.VMEM((CH * D,), jnp.float32),
            pltpu.VMEM((CH * D,), jnp.float32),
            pltpu.VMEM((CH * D,), jnp.float32),
            pltpu.VMEM((CH * D,), jnp.float32),
            pltpu.VMEM((128,), jnp.int32),
            pltpu.VMEM((TOK,), jnp.int32),
            pltpu.VMEM((TOK,), jnp.int32),
            pltpu.VMEM((128,), jnp.float32),
            pltpu.VMEM((G * T,), jnp.float32),
            pltpu.VMEM((128,), jnp.float32),
            pltpu.VMEM((NTPB * 128,), jnp.float32),
            pltpu.VMEM((HB // NC * 2 * T,), jnp.float32),
            pltpu.VMEM((2 * T,), jnp.float32),
            pltpu.VMEM((128,), jnp.float32),
            pltpu.VMEM_SHARED((NS * 128,), jnp.float32),
            pltpu.VMEM_SHARED((HB // NC * 2 * T,), jnp.float32),
            pltpu.SemaphoreType.DMA,
            pltpu.SemaphoreType.DMA,
            pltpu.SemaphoreType.DMA,
            pltpu.SemaphoreType.DMA,
        ],
    )(functools.partial(_sc_main, B=B, N=N, D=D, G=G, T=T,
                        HB=HB, NTPB=NTPB, TOK=TOK, CH=CH, NCH=NCH))(
        pred1, target1, idx_pad, it2)

    out = pl.pallas_call(
        functools.partial(_comb_body, T=T),
        in_specs=[
            pl.BlockSpec((1, 2 * T), lambda: (0, 0)),
            pl.BlockSpec((NC, 2 * T), lambda: (0, 0)),
        ],
        out_specs=pl.BlockSpec((1, T), lambda: (0, 0)),
        out_shape=jax.ShapeDtypeStruct((1, T), jnp.float32),
    )(tc_part, sc_part)
    return out.reshape(T)


# speedup vs baseline: 1.0213x; 1.0213x over previous
import functools

import jax
import jax.numpy as jnp
from jax import lax
from jax.experimental import pallas as pl
from jax.experimental.pallas import tpu as pltpu
from jax.experimental.pallas import tpu_sc as plsc

NC = 2
NS = 16
LANES = 16


def _tc_body(pred_ref, target_ref, s_ref, e_ref, out_ref, *, X, D, G):
    p = pred_ref[0]
    t = target_ref[0]
    d = p - t
    err = d * d
    ones_d = jnp.ones((D, 1), jnp.float32)
    tok = jax.lax.dot_general(
        err, ones_d, (((1,), (0,)), ((), ())),
        preferred_element_type=jnp.float32,
        precision=jax.lax.Precision.HIGHEST) * (1.0 / D)

    s = s_ref[0]
    e = e_ref[0]
    n_iota = jax.lax.broadcasted_iota(jnp.int32, (G, X), 1)
    mask = ((n_iota >= s) & (n_iota < e)).astype(jnp.float32)
    gsum = jax.lax.dot_general(
        mask, tok, (((1,), (0,)), ((), ())),
        preferred_element_type=jnp.float32,
        precision=jax.lax.Precision.HIGHEST)
    out_ref[0] = gsum


def _sc_main(pred_hbm, target_hbm, idx_hbm, out_hbm,
             pbuf0, pbuf1, tbuf0, tbuf1, idx_v, bcnt_v, gmap_v, gsum_v,
             row2_v, out_v, bst_v, shared_gsum,
             sp0, sp1, st0, st1,
             *, B, N, D, G, X, TOK, CH, NCH):
    c = lax.axis_index("c")
    s = lax.axis_index("s")
    wid = c * NS + s
    b = wid // 2
    pos_base = X + (wid % 2) * TOK
    lanes = lax.iota(jnp.int32, LANES)
    zero16i = jnp.zeros((LANES,), jnp.int32)
    zero16f = jnp.zeros((LANES,), jnp.float32)

    CHW = CH * D

    def _issue(ch, pb, tb, sp, st):
        off = (b * N + pos_base + ch * CH) * D
        pltpu.async_copy(pred_hbm.at[pl.ds(off, CHW)], pb, sp)
        pltpu.async_copy(target_hbm.at[pl.ds(off, CHW)], tb, st)

    _issue(0, pbuf0, tbuf0, sp0, st0)
    _issue(1, pbuf1, tbuf1, sp1, st1)

    pltpu.sync_copy(idx_hbm.at[b], idx_v)

    def _zb(i, carry):
        bcnt_v[pl.ds(i * LANES, LANES)] = zero16i
        return carry
    lax.fori_loop(0, TOK // LANES, _zb, 0)

    ones16i = jnp.ones((LANES,), jnp.int32)
    for v in range((G + LANES) // LANES):
        j = v * LANES + lanes
        bv = idx_v[pl.ds(v * LANES, LANES)]
        rel = bv - pos_base
        valid = (j <= G) & (rel < TOK)
        tgt = jnp.maximum(rel, 0)
        plsc.addupdate_scatter(bcnt_v, [tgt], ones16i, mask=valid)

    def _cs(i, carry):
        x = bcnt_v[pl.ds(i * LANES, LANES)]
        cs = plsc.cumsum(x) + carry
        g = cs - 1
        g = jnp.where((g >= 0) & (g < G), g, 127)
        gmap_v[pl.ds(i * LANES, LANES)] = g
        return carry + jnp.sum(x)
    lax.fori_loop(0, TOK // LANES, _cs, jnp.int32(0))

    for k in range(128 // LANES):
        gsum_v[pl.ds(k * LANES, LANES)] = zero16f

    def _wait(pb, tb, sp, st):
        pltpu.make_async_copy(pred_hbm.at[pl.ds(0, CHW)], pb, sp).wait()
        pltpu.make_async_copy(target_hbm.at[pl.ds(0, CHW)], tb, st).wait()

    shuf_dn = lax.GatherDimensionNumbers(
        offset_dims=(), collapsed_slice_dims=(0,), start_index_map=(0,))

    def _shuf(v, idx):
        return lax.gather(v, idx[:, None], shuf_dn, slice_sizes=(1,),
                          mode=lax.GatherScatterMode.PROMISE_IN_BOUNDS)

    xor_idx = {o: lanes ^ o for o in (8, 4, 2, 1)}
    xor_msk = {o: (lanes & o) == 0 for o in (8, 4, 2, 1)}
    pi_vec = (((lanes & 1) << 3) | ((lanes & 2) << 1) |
              ((lanes & 4) >> 1) | ((lanes & 8) >> 3))

    def _combine(x, y, o):
        return jnp.where(xor_msk[o],
                         x + _shuf(x, xor_idx[o]),
                         y + _shuf(y, xor_idx[o]))

    def _fold(pb, tb, off):
        sqs = []
        for v in range(D // LANES):
            pv = pb[pl.ds(off + v * LANES, LANES)]
            tv = tb[pl.ds(off + v * LANES, LANES)]
            dv = pv - tv
            sqs.append(dv * dv)
        while len(sqs) > 1:
            sqs = [sqs[2 * k] + sqs[2 * k + 1] for k in range(len(sqs) // 2)]
        return sqs[0]

    def _compute(pb, tb, ch):
        def _tg(tg, carry):
            for j in range(LANES // 2):
                off = (tg * LANES + 2 * j) * D
                a0 = _fold(pb, tb, off)
                a1 = _fold(pb, tb, off + D)
                bst_v[pl.ds(j * LANES, LANES)] = _combine(a0, a1, 8)
            bs_ = [bst_v[pl.ds(j * LANES, LANES)] for j in range(LANES // 2)]
            cs_ = [_combine(bs_[2 * k], bs_[2 * k + 1], 4) for k in range(4)]
            ds_ = [_combine(cs_[2 * k], cs_[2 * k + 1], 2) for k in range(2)]
            te = _combine(ds_[0], ds_[1], 1)
            g = gmap_v[pl.ds(ch * CH + tg * LANES, LANES)]
            gp = _shuf(g, pi_vec)
            plsc.addupdate_scatter(gsum_v, [gp], te)
            return carry
        lax.fori_loop(0, CH // LANES, _tg, 0)

    def _pair(i, carry):
        ch0 = 2 * i
        _wait(pbuf0, tbuf0, sp0, st0)
        _compute(pbuf0, tbuf0, ch0)

        @pl.when(ch0 + 2 < NCH)
        def _():
            _issue(ch0 + 2, pbuf0, tbuf0, sp0, st0)

        _wait(pbuf1, tbuf1, sp1, st1)
        _compute(pbuf1, tbuf1, ch0 + 1)

        @pl.when(ch0 + 3 < NCH)
        def _():
            _issue(ch0 + 3, pbuf1, tbuf1, sp1, st1)
        return carry
    lax.fori_loop(0, NCH // 2, _pair, 0)

    pltpu.sync_copy(gsum_v, shared_gsum.at[pl.ds(s * 128, 128)])
    plsc.subcore_barrier()

    @pl.when(s < NS // 2)
    def _():
        bg = c * (NS // 2) + s
        pltpu.sync_copy(shared_gsum.at[pl.ds((2 * s) * 128, 256)], row2_v)
        for j in range(G // LANES):
            out_v[pl.ds(j * LANES, LANES)] = (
                row2_v[pl.ds(j * LANES, LANES)]
                + row2_v[pl.ds(128 + j * LANES, LANES)])
        pltpu.sync_copy(out_v, out_hbm.at[bg])


def _route_body(tcg_ref, scg_ref, s_ref, e_ref, it_ref, out_ref, *, D, T):
    tc_g = tcg_ref[...]
    sc_g = scg_ref[...]
    s = s_ref[...]
    e = e_ref[...]
    cnt = (e - s).astype(jnp.float32)
    ge = (tc_g + sc_g * (1.0 / D)) / jnp.maximum(cnt, 1.0)

    it = it_ref[...]
    maxv = jnp.max(it, axis=1, keepdims=True)
    colidx = jax.lax.broadcasted_iota(jnp.int32, it.shape, 1)
    am = jnp.min(jnp.where(it == maxv, colidx, T), axis=1, keepdims=True)
    onehot = (colidx == am).astype(jnp.float32)

    tsum = jnp.sum(onehot * ge, axis=0, keepdims=True)
    tcnt = jnp.sum(onehot, axis=0, keepdims=True)
    out_ref[...] = jnp.where(tcnt > 0, tsum / jnp.maximum(tcnt, 1.0), 0.0)


def kernel(pred, target, indices, indices_type, type_names):
    B, N, D = pred.shape
    G = indices.shape[1] - 1
    T = indices_type.shape[2]
    X = 1536
    TOK = (N - X) // 2
    CH = 128
    NCH = TOK // CH

    starts = indices[:, :-1]
    ends = indices[:, 1:]

    tc_gsum = pl.pallas_call(
        functools.partial(_tc_body, X=X, D=D, G=G),
        grid=(B,),
        in_specs=[
            pl.BlockSpec((1, X, D), lambda b: (b, 0, 0)),
            pl.BlockSpec((1, X, D), lambda b: (b, 0, 0)),
            pl.BlockSpec((1, G, 1), lambda b: (b, 0, 0)),
            pl.BlockSpec((1, G, 1), lambda b: (b, 0, 0)),
        ],
        out_specs=pl.BlockSpec((1, G, 1), lambda b: (b, 0, 0)),
        out_shape=jax.ShapeDtypeStruct((B, G, 1), jnp.float32),
    )(pred, target, starts[:, :, None], ends[:, :, None])

    pred1 = pred.reshape(B * N * D)
    target1 = target.reshape(B * N * D)
    idx_pad = jnp.pad(indices, ((0, 0), (0, 128 - (G + 1))))

    mesh = plsc.VectorSubcoreMesh(core_axis_name="c", subcore_axis_name="s")
    params = pltpu.CompilerParams(needs_layout_passes=False)
    sc_gsum = functools.partial(
        pl.kernel,
        out_type=jax.ShapeDtypeStruct((B, G), jnp.float32),
        mesh=mesh,
        compiler_params=params,
        scratch_types=[
            pltpu.VMEM((CH * D,), jnp.float32),
            pltpu.VMEM((CH * D,), jnp.float32),
            pltpu.VMEM((CH * D,), jnp.float32),
            pltpu.VMEM((CH * D,), jnp.float32),
            pltpu.VMEM((128,), jnp.int32),
            pltpu.VMEM((TOK,), jnp.int32),
            pltpu.VMEM((TOK,), jnp.int32),
            pltpu.VMEM((128,), jnp.float32),
            pltpu.VMEM((256,), jnp.float32),
            pltpu.VMEM((G,), jnp.float32),
            pltpu.VMEM((128,), jnp.float32),
            pltpu.VMEM_SHARED((NS * 128,), jnp.float32),
            pltpu.SemaphoreType.DMA,
            pltpu.SemaphoreType.DMA,
            pltpu.SemaphoreType.DMA,
            pltpu.SemaphoreType.DMA,
        ],
    )(functools.partial(_sc_main, B=B, N=N, D=D, G=G,
                        X=X, TOK=TOK, CH=CH, NCH=NCH))(
        pred1, target1, idx_pad)

    BG = B * G
    out = pl.pallas_call(
        functools.partial(_route_body, D=D, T=T),
        in_specs=[
            pl.BlockSpec((BG, 1), lambda: (0, 0)),
            pl.BlockSpec((BG, 1), lambda: (0, 0)),
            pl.BlockSpec((BG, 1), lambda: (0, 0)),
            pl.BlockSpec((BG, 1), lambda: (0, 0)),
            pl.BlockSpec((BG, T), lambda: (0, 0)),
        ],
        out_specs=pl.BlockSpec((1, T), lambda: (0, 0)),
        out_shape=jax.ShapeDtypeStruct((1, T), jnp.float32),
    )(tc_gsum.reshape(BG, 1), sc_gsum.reshape(BG, 1),
      starts.reshape(BG, 1), ends.reshape(BG, 1),
      indices_type.reshape(BG, T))
    return out.reshape(T)
